# baseline (speedup 1.0000x reference)
import functools

import jax
import jax.numpy as jnp
from jax import lax
from jax.experimental import pallas as pl
from jax.experimental.pallas import tpu as pltpu
from jax.experimental.pallas import tpu_sc as plsc

_VOCAB_SIZE = 1_000_000
_EMBED_DIM = 32
_NUM_NEG = 5
_NUM_CORES = 2
_NUM_SUBCORES = 16
_NUM_WORKERS = _NUM_CORES * _NUM_SUBCORES


def _sc_gather(center_table, context_table, center_idx, neg_idx):
    b = center_idx.shape[0]
    n = neg_idx.shape[0]
    bc = b // _NUM_WORKERS
    bn = n // _NUM_WORKERS
    mesh = plsc.VectorSubcoreMesh(core_axis_name="c", subcore_axis_name="s")

    @functools.partial(
        pl.kernel,
        mesh=mesh,
        compiler_params=pltpu.CompilerParams(use_tc_tiling_on_sc=False),
        out_type=(
            jax.ShapeDtypeStruct((b, _EMBED_DIM), jnp.float32),
            jax.ShapeDtypeStruct((n, _EMBED_DIM), jnp.float32),
        ),
        scratch_types=[
            pltpu.VMEM((bc,), jnp.int32),
            pltpu.VMEM((bc, _EMBED_DIM), jnp.float32),
            pltpu.VMEM((bn,), jnp.int32),
            pltpu.VMEM((bn, _EMBED_DIM), jnp.float32),
            pltpu.SemaphoreType.DMA,
            pltpu.SemaphoreType.DMA,
        ],
    )
    def gather_kernel(ct_hbm, xt_hbm, ci_hbm, ni_hbm, co_hbm, no_hbm,
                      ci_v, crows_v, ni_v, nrows_v, csem, nsem):
        wid = lax.axis_index("s") * _NUM_CORES + lax.axis_index("c")
        cbase = wid * bc
        nbase = wid * bn
        pltpu.sync_copy(ci_hbm.at[pl.ds(cbase, bc)], ci_v)
        pltpu.sync_copy(ni_hbm.at[pl.ds(nbase, bn)], ni_v)
        ccopy = pltpu.async_copy(ct_hbm.at[ci_v], crows_v, csem)
        ncopy = pltpu.async_copy(xt_hbm.at[ni_v], nrows_v, nsem)
        ccopy.wait()
        pltpu.sync_copy(crows_v, co_hbm.at[pl.ds(cbase, bc)])
        ncopy.wait()
        pltpu.sync_copy(nrows_v, no_hbm.at[pl.ds(nbase, bn)])

    return gather_kernel(center_table, context_table, center_idx, neg_idx)


def _log_sigmoid(x):
    return jnp.where(x >= 0, -jnp.log1p(jnp.exp(-x)), x - jnp.log1p(jnp.exp(x)))


def _tc_loss(pos2d, ce, ne):
    b = ce.shape[0]
    total = b * _NUM_NEG

    def loss_body(pos_ref, ce_ref, ne_ref, out_ref):
        pos = pos_ref[...]
        pos_acc = jnp.sum(_log_sigmoid(pos)) / b
        cew = ce_ref[...]
        neg_acc = jnp.float32(0.0)
        for k in range(_NUM_NEG):
            nek = ne_ref[pl.ds(k * b, b), :]
            s = jnp.sum(cew * nek, axis=1, keepdims=True)
            neg_acc += jnp.sum(_log_sigmoid(-s))
        out_ref[0, 0] = -pos_acc - neg_acc / total

    return pl.pallas_call(
        loss_body,
        out_shape=jax.ShapeDtypeStruct((1, 1), jnp.float32),
        out_specs=pl.BlockSpec(memory_space=pltpu.SMEM),
    )(pos2d, ce, ne)


def kernel(pos_scores, center_words, center_table, context_table):
    batch = pos_scores.shape[0]
    center_idx = center_words.astype(jnp.int32)
    neg_words = jax.random.randint(
        jax.random.key(42), (batch, _NUM_NEG), 0, _VOCAB_SIZE)
    neg_idx = neg_words.T.reshape(-1).astype(jnp.int32)
    ce, ne = _sc_gather(center_table, context_table, center_idx, neg_idx)
    pos2d = pos_scores.reshape(128, -1)
    loss = _tc_loss(pos2d, ce, ne)
    return loss.reshape(())
